# baseline (speedup 1.0000x reference)
import functools

import jax
import jax.numpy as jnp
from jax import lax
from jax.experimental import pallas as pl
from jax.experimental.pallas import tpu as pltpu
from jax.experimental.pallas import tpu_sc as plsc

VOCAB = 1000000
EMBED_DIM = 64
PADDED_DIM = 128
BATCH = 4096
HIST = 200

NUM_CORES = 2
NUM_SUBCORES = 16
NUM_WORKERS = NUM_CORES * NUM_SUBCORES
LANES = 16

TBLOCK = 400
NTBLOCKS = VOCAB // TBLOCK
NTB_W = NTBLOCKS // NUM_WORKERS
NTB_EXTRA = NTBLOCKS - NTB_W * NUM_WORKERS
NFULL = TBLOCK // LANES

XROWS = BATCH // NUM_WORKERS
NUM_CHUNKS = XROWS
NBUF = 4
DEPTH = NBUF - 1


def _transpose_kernel(tab_t_hbm, out_hbm, in_v0, in_v1, out_v0, out_v1,
                      isems, osems):
    wid = lax.axis_index("s") * NUM_CORES + lax.axis_index("c")
    wblk0 = wid * NTB_W
    lane_iota = lax.iota(jnp.int32, LANES)
    cols = [jnp.full((LANES,), f, jnp.int32) for f in range(EMBED_DIM)]
    in_bufs = (in_v0, in_v1)
    out_bufs = (out_v0, out_v1)

    def in_copy(blk, s):
        src = tab_t_hbm.at[:, pl.ds(pl.multiple_of(blk * TBLOCK, TBLOCK), TBLOCK)]
        return pltpu.make_async_copy(src, in_bufs[s], isems[s])

    def out_copy(blk, s):
        dst = out_hbm.at[pl.ds(blk * TBLOCK, TBLOCK)]
        return pltpu.make_async_copy(out_bufs[s], dst, osems[s])

    def transpose_block(s):
        def chunk_loop(j, c2):
            b0 = j * LANES
            rows = b0 + lane_iota
            for f in range(EMBED_DIM):
                v = in_bufs[s][f, pl.ds(b0, LANES)]
                plsc.store_scatter(out_bufs[s], [rows, cols[f]], v)
            return c2
        lax.fori_loop(0, NFULL, chunk_loop, 0)

    def step(t, s):
        in_copy(wblk0 + t, s).wait()

        @pl.when(t >= 2)
        def _():
            out_copy(wblk0 + t - 2, s).wait()
        transpose_block(s)
        out_copy(wblk0 + t, s).start()

        @pl.when(t + 2 < NTB_W)
        def _():
            in_copy(wblk0 + t + 2, s).start()

    in_copy(wblk0 + 0, 0).start()
    in_copy(wblk0 + 1, 1).start()

    def pair(p, carry):
        for s in range(2):
            step(2 * p + s, s)
        return carry

    lax.fori_loop(0, NTB_W // 2, pair, 0)

    out_copy(wblk0 + NTB_W - 2, 0).wait()
    out_copy(wblk0 + NTB_W - 1, 1).wait()

    @pl.when(wid < NTB_EXTRA)
    def _():
        blk = NTB_W * NUM_WORKERS + wid
        in_copy(blk, 0).start()
        in_copy(blk, 0).wait()
        transpose_block(0)
        out_copy(blk, 0).start()
        out_copy(blk, 0).wait()


def _gather_kernel(x_hbm, table_hbm, out_hbm, idx_all, rows_v, gsems, ssems):
    wid = lax.axis_index("s") * NUM_CORES + lax.axis_index("c")
    wrow = wid * XROWS

    pltpu.sync_copy(x_hbm.at[pl.ds(wrow, XROWS)], idx_all)

    def gather_copy(c, b):
        src = table_hbm.at[idx_all.at[c]]
        return pltpu.make_async_copy(src, rows_v.at[b], gsems[b])

    def store_copy(c, b):
        dst = out_hbm.at[wrow + c, :, pl.ds(0, EMBED_DIM)]
        return pltpu.make_async_copy(rows_v.at[b], dst, ssems[b])

    def visit(c, b):
        gather_copy(c, b).wait()
        store_copy(c, b).start()
        h = c + DEPTH
        hb = (b + DEPTH) % NBUF

        @pl.when(h < NUM_CHUNKS)
        def _():
            @pl.when(h >= NBUF)
            def _():
                store_copy(h - NBUF, hb).wait()
            gather_copy(h, hb).start()

    for h in range(DEPTH):
        gather_copy(h, h).start()

    def group(p, carry):
        for b in range(NBUF):
            visit(NBUF * p + b, b)
        return carry

    lax.fori_loop(0, NUM_CHUNKS // NBUF, group, 0)

    for b in range(NBUF):
        store_copy(NUM_CHUNKS - NBUF + b, b).wait()


@jax.jit
def _embed(x, table):
    mesh = plsc.VectorSubcoreMesh(
        core_axis_name="c", subcore_axis_name="s",
        num_cores=NUM_CORES, num_subcores=NUM_SUBCORES,
    )
    params = pltpu.CompilerParams(use_tc_tiling_on_sc=False)
    tparams = pltpu.CompilerParams(
        use_tc_tiling_on_sc=False, needs_layout_passes=False)

    transpose = functools.partial(
        pl.kernel,
        out_type=jax.ShapeDtypeStruct((VOCAB, EMBED_DIM), jnp.float32),
        mesh=mesh,
        scratch_types=[
            pltpu.VMEM((EMBED_DIM, TBLOCK), jnp.float32),
            pltpu.VMEM((EMBED_DIM, TBLOCK), jnp.float32),
            pltpu.VMEM((TBLOCK, EMBED_DIM), jnp.float32),
            pltpu.VMEM((TBLOCK, EMBED_DIM), jnp.float32),
            [pltpu.SemaphoreType.DMA] * 2,
            [pltpu.SemaphoreType.DMA] * 2,
        ],
        compiler_params=tparams,
    )(_transpose_kernel)
    table_lin = transpose(table.T)

    gather = functools.partial(
        pl.kernel,
        out_type=jax.ShapeDtypeStruct((BATCH, HIST, PADDED_DIM), jnp.float32),
        mesh=mesh,
        scratch_types=[
            pltpu.VMEM((XROWS, HIST), jnp.int32),
            pltpu.VMEM((NBUF, HIST, EMBED_DIM), jnp.float32),
            [pltpu.SemaphoreType.DMA] * NBUF,
            [pltpu.SemaphoreType.DMA] * NBUF,
        ],
        compiler_params=params,
    )(_gather_kernel)
    out_p = gather(x, table_lin)
    return out_p[:, :, :EMBED_DIM]


def kernel(x, table):
    return _embed(x, table)


# speedup vs baseline: 1.1016x; 1.1016x over previous
import functools

import jax
import jax.numpy as jnp
from jax import lax
from jax.experimental import layout as jax_layout
from jax.experimental import pallas as pl
from jax.experimental.pallas import tpu as pltpu
from jax.experimental.pallas import tpu_sc as plsc

VOCAB = 1000000
EMBED_DIM = 64
PADDED_DIM = 128
BATCH = 4096
HIST = 200

NUM_CORES = 2
NUM_SUBCORES = 16
NUM_WORKERS = NUM_CORES * NUM_SUBCORES
LANES = 16

TBLOCK = 400
NTBLOCKS = VOCAB // TBLOCK
NTB_W = NTBLOCKS // NUM_WORKERS
NTB_EXTRA = NTBLOCKS - NTB_W * NUM_WORKERS
NFULL = TBLOCK // LANES

XROWS = BATCH // NUM_WORKERS
NUM_CHUNKS = XROWS
NBUF = 4
DEPTH = NBUF - 1


def _transpose_kernel(tab_t_hbm, out_hbm, in_v0, in_v1, out_v0, out_v1,
                      isems, osems):
    wid = lax.axis_index("s") * NUM_CORES + lax.axis_index("c")
    wblk0 = wid * NTB_W
    lane_iota = lax.iota(jnp.int32, LANES)
    cols = [jnp.full((LANES,), f, jnp.int32) for f in range(EMBED_DIM)]
    in_bufs = (in_v0, in_v1)
    out_bufs = (out_v0, out_v1)

    def in_copy(blk, s):
        src = tab_t_hbm.at[:, pl.ds(pl.multiple_of(blk * TBLOCK, TBLOCK), TBLOCK)]
        return pltpu.make_async_copy(src, in_bufs[s], isems[s])

    def out_copy(blk, s):
        dst = out_hbm.at[pl.ds(blk * TBLOCK, TBLOCK)]
        src = out_bufs[s].at[:, pl.ds(0, EMBED_DIM)]
        return pltpu.make_async_copy(src, dst, osems[s])

    def transpose_block(s):
        def chunk_loop(j, c2):
            b0 = j * LANES
            rows = b0 + lane_iota
            for f in range(EMBED_DIM):
                v = in_bufs[s][f, pl.ds(b0, LANES)]
                plsc.store_scatter(out_bufs[s], [rows, cols[f]], v)
            return c2
        lax.fori_loop(0, NFULL, chunk_loop, 0)

    def step(t, s):
        in_copy(wblk0 + t, s).wait()

        @pl.when(t >= 2)
        def _():
            out_copy(wblk0 + t - 2, s).wait()
        transpose_block(s)
        out_copy(wblk0 + t, s).start()

        @pl.when(t + 2 < NTB_W)
        def _():
            in_copy(wblk0 + t + 2, s).start()

    in_copy(wblk0 + 0, 0).start()
    in_copy(wblk0 + 1, 1).start()

    def pair(p, carry):
        for s in range(2):
            step(2 * p + s, s)
        return carry

    lax.fori_loop(0, NTB_W // 2, pair, 0)

    out_copy(wblk0 + NTB_W - 2, 0).wait()
    out_copy(wblk0 + NTB_W - 1, 1).wait()

    @pl.when(wid < NTB_EXTRA)
    def _():
        blk = NTB_W * NUM_WORKERS + wid
        in_copy(blk, 0).start()
        in_copy(blk, 0).wait()
        transpose_block(0)
        out_copy(blk, 0).start()
        out_copy(blk, 0).wait()


def _gather_kernel(x_hbm, table_hbm, out_hbm, idx_all, rows_v, gsems, ssems):
    wid = lax.axis_index("s") * NUM_CORES + lax.axis_index("c")
    wrow = wid * XROWS

    pltpu.sync_copy(x_hbm.at[pl.ds(wrow, XROWS)], idx_all)

    def gather_copy(c, b):
        src = table_hbm.at[idx_all.at[c]]
        return pltpu.make_async_copy(src, rows_v.at[b], gsems[b])

    def store_copy(c, b):
        dst = out_hbm.at[wrow + c, :, pl.ds(0, EMBED_DIM)]
        return pltpu.make_async_copy(rows_v.at[b], dst, ssems[b])

    def visit(c, b):
        gather_copy(c, b).wait()
        store_copy(c, b).start()
        h = c + DEPTH
        hb = (b + DEPTH) % NBUF

        @pl.when(h < NUM_CHUNKS)
        def _():
            @pl.when(h >= NBUF)
            def _():
                store_copy(h - NBUF, hb).wait()
            gather_copy(h, hb).start()

    for h in range(DEPTH):
        gather_copy(h, h).start()

    def group(p, carry):
        for b in range(NBUF):
            visit(NBUF * p + b, b)
        return carry

    lax.fori_loop(0, NUM_CHUNKS // NBUF, group, 0)

    for b in range(NBUF):
        store_copy(NUM_CHUNKS - NBUF + b, b).wait()


@jax.jit
def _embed(x, table):
    mesh = plsc.VectorSubcoreMesh(
        core_axis_name="c", subcore_axis_name="s",
        num_cores=NUM_CORES, num_subcores=NUM_SUBCORES,
    )
    params = pltpu.CompilerParams(use_tc_tiling_on_sc=False)
    tparams = pltpu.CompilerParams(
        use_tc_tiling_on_sc=False, needs_layout_passes=False)

    transpose = functools.partial(
        pl.kernel,
        out_type=jax.ShapeDtypeStruct((VOCAB, EMBED_DIM), jnp.float32),
        mesh=mesh,
        scratch_types=[
            pltpu.VMEM((EMBED_DIM, TBLOCK), jnp.float32),
            pltpu.VMEM((EMBED_DIM, TBLOCK), jnp.float32),
            pltpu.VMEM((TBLOCK, EMBED_DIM + 1), jnp.float32),
            pltpu.VMEM((TBLOCK, EMBED_DIM + 1), jnp.float32),
            [pltpu.SemaphoreType.DMA] * 2,
            [pltpu.SemaphoreType.DMA] * 2,
        ],
        compiler_params=tparams,
    )(_transpose_kernel)
    table_t = jax_layout.with_layout_constraint(
        table.T, jax_layout.Layout((1, 0)))
    table_lin = transpose(table_t)

    gather = functools.partial(
        pl.kernel,
        out_type=jax.ShapeDtypeStruct((BATCH, HIST, PADDED_DIM), jnp.float32),
        mesh=mesh,
        scratch_types=[
            pltpu.VMEM((XROWS, HIST), jnp.int32),
            pltpu.VMEM((NBUF, HIST, EMBED_DIM), jnp.float32),
            [pltpu.SemaphoreType.DMA] * NBUF,
            [pltpu.SemaphoreType.DMA] * NBUF,
        ],
        compiler_params=params,
    )(_gather_kernel)
    out_p = gather(x, table_lin)
    return out_p[:, :, :EMBED_DIM]


def kernel(x, table):
    return _embed(x, table)


# speedup vs baseline: 7.0102x; 6.3636x over previous
import functools

import jax
import jax.numpy as jnp
from jax import lax
from jax.experimental import pallas as pl
from jax.experimental.pallas import tpu as pltpu
from jax.experimental.pallas import tpu_sc as plsc

VOCAB = 1000000
EMBED_DIM = 64
PADDED_DIM = 128
BATCH = 4096
HIST = 200

NUM_CORES = 2
NUM_SUBCORES = 16
NUM_WORKERS = NUM_CORES * NUM_SUBCORES

XROWS = BATCH // NUM_WORKERS
NUM_CHUNKS = XROWS
NBUF = 4
DEPTH = NBUF - 1


def _gather_kernel(x_hbm, table_hbm, out_hbm, idx_all, rows_v, gsems, ssems):
    wid = lax.axis_index("s") * NUM_CORES + lax.axis_index("c")
    wrow = wid * XROWS

    pltpu.sync_copy(x_hbm.at[pl.ds(wrow, XROWS)], idx_all)

    def gather_copy(c, b):
        src = table_hbm.at[idx_all.at[c]]
        return pltpu.make_async_copy(src, rows_v.at[b], gsems[b])

    def store_copy(c, b):
        dst = out_hbm.at[wrow + c, :, pl.ds(0, EMBED_DIM)]
        src = rows_v.at[b, :, pl.ds(0, EMBED_DIM)]
        return pltpu.make_async_copy(src, dst, ssems[b])

    def visit(c, b):
        gather_copy(c, b).wait()
        store_copy(c, b).start()
        h = c + DEPTH
        hb = (b + DEPTH) % NBUF

        @pl.when(h < NUM_CHUNKS)
        def _():
            @pl.when(h >= NBUF)
            def _():
                store_copy(h - NBUF, hb).wait()
            gather_copy(h, hb).start()

    for h in range(DEPTH):
        gather_copy(h, h).start()

    def group(p, carry):
        for b in range(NBUF):
            visit(NBUF * p + b, b)
        return carry

    lax.fori_loop(0, NUM_CHUNKS // NBUF, group, 0)

    for b in range(NBUF):
        store_copy(NUM_CHUNKS - NBUF + b, b).wait()


@jax.jit
def _embed(x, table):
    table_p = jnp.pad(table, ((0, 0), (0, PADDED_DIM - EMBED_DIM)))
    mesh = plsc.VectorSubcoreMesh(
        core_axis_name="c", subcore_axis_name="s",
        num_cores=NUM_CORES, num_subcores=NUM_SUBCORES,
    )
    run = functools.partial(
        pl.kernel,
        out_type=jax.ShapeDtypeStruct((BATCH, HIST, PADDED_DIM), jnp.float32),
        mesh=mesh,
        scratch_types=[
            pltpu.VMEM((XROWS, HIST), jnp.int32),
            pltpu.VMEM((NBUF, HIST, PADDED_DIM), jnp.float32),
            [pltpu.SemaphoreType.DMA] * NBUF,
            [pltpu.SemaphoreType.DMA] * NBUF,
        ],
        compiler_params=pltpu.CompilerParams(use_tc_tiling_on_sc=False),
    )(_gather_kernel)
    out_p = run(x, table_p)
    return out_p[:, :, :EMBED_DIM]


def kernel(x, table):
    return _embed(x, table)
